# baseline (speedup 1.0000x reference)
import functools

import jax
import jax.numpy as jnp
from jax import lax
from jax.experimental import pallas as pl
from jax.experimental.pallas import tpu as pltpu
from jax.experimental.pallas import tpu_sc as plsc

NUM_STUDENTS = 100000
NUM_QUESTIONS = 128
NUM_CONCEPTS = 128
GUESS_PROB = 0.2
L = 10.0

LANES = 16
NUM_CORES = 2
NUM_SUBCORES = 16
NUM_WORKERS = NUM_CORES * NUM_SUBCORES
ROWS_PER_WORKER = NUM_STUDENTS // NUM_WORKERS
CHUNK = 125
NUM_CHUNKS = ROWS_PER_WORKER // CHUNK
NUM_PAIRS = NUM_CHUNKS // 2
GROUPS = NUM_QUESTIONS // LANES
CHUNK_ELEMS = CHUNK * NUM_QUESTIONS


def _fe_body(A_hbm, D_hbm, row_hbm, col_hbm, out_hbm,
             in0, in1, out0, out1, col_v, row_v, d_v, D_v,
             si0, si1, so0, so1):
    wid = lax.axis_index("s") * NUM_CORES + lax.axis_index("c")
    base = wid * ROWS_PER_WORKER

    def a_slice(ci):
        return A_hbm.at[pl.ds((base + ci * CHUNK) * NUM_QUESTIONS,
                              CHUNK_ELEMS)]

    def o_slice(ci):
        return out_hbm.at[pl.ds((base + ci * CHUNK) * NUM_QUESTIONS,
                                CHUNK_ELEMS)]

    def start_in(ci, buf, sem):
        pltpu.async_copy(a_slice(ci), buf, sem)

    def wait_in(ci, buf, sem):
        pltpu.make_async_copy(a_slice(ci), buf, sem).wait()

    def start_out(ci, buf, sem):
        pltpu.async_copy(buf, o_slice(ci), sem)

    def wait_out(ci, buf, sem):
        pltpu.make_async_copy(buf, o_slice(ci), sem).wait()

    start_in(0, in0, si0)

    pltpu.sync_copy(col_hbm, col_v)
    pltpu.sync_copy(row_hbm, row_v)
    pltpu.sync_copy(D_hbm, D_v)

    for g in range(GROUPS):
        sl = pl.ds(g * LANES, LANES)
        d_v[sl] = plsc.load_gather(
            D_v, [row_v[sl] * NUM_CONCEPTS + col_v[sl]])

    col_g = [col_v[pl.ds(g * LANES, LANES)] for g in range(GROUPS)]
    d10_g = [d_v[pl.ds(g * LANES, LANES)] * jnp.float32(L)
             for g in range(GROUPS)]

    def compute(in_buf, out_buf):
        @plsc.parallel_loop(0, CHUNK, unroll=4)
        def row_body(s):
            s_base = s * NUM_QUESTIONS
            for g in range(GROUPS):
                a = plsc.load_gather(in_buf, [col_g[g] + s_base])
                t = jnp.exp(a * jnp.float32(-L) + d10_g[g])
                out_buf[pl.ds(s_base + g * LANES, LANES)] = jnp.maximum(
                    1.0 - t, jnp.float32(GUESS_PROB))

    start_in(1, in1, si1)
    wait_in(0, in0, si0)
    compute(in0, out0)
    start_out(0, out0, so0)
    start_in(2, in0, si0)
    wait_in(1, in1, si1)
    compute(in1, out1)
    start_out(1, out1, so1)

    def pair_body(k, carry):
        c0 = 2 * k
        c1 = c0 + 1
        start_in(c1, in1, si1)
        wait_in(c0, in0, si0)
        wait_out(c0 - 2, out0, so0)
        compute(in0, out0)
        start_out(c0, out0, so0)
        start_in(c0 + 2, in0, si0)
        wait_in(c1, in1, si1)
        wait_out(c1 - 2, out1, so1)
        compute(in1, out1)
        start_out(c1, out1, so1)
        return carry

    lax.fori_loop(1, NUM_PAIRS, pair_body, 0)

    last = NUM_CHUNKS - 1
    wait_in(last, in0, si0)
    wait_out(last - 2, out0, so0)
    compute(in0, out0)
    start_out(last, out0, so0)
    wait_out(last - 1, out1, so1)
    wait_out(last, out0, so0)


@jax.jit
def _fe_model(A, D, concepts_row, concepts_col):
    mesh = plsc.VectorSubcoreMesh(core_axis_name="c", subcore_axis_name="s")
    run = functools.partial(
        pl.kernel,
        mesh=mesh,
        compiler_params=pltpu.CompilerParams(needs_layout_passes=False),
        out_type=jax.ShapeDtypeStruct((NUM_STUDENTS * NUM_QUESTIONS,),
                                      jnp.float32),
        scratch_types=[
            pltpu.VMEM((CHUNK_ELEMS,), jnp.float32),
            pltpu.VMEM((CHUNK_ELEMS,), jnp.float32),
            pltpu.VMEM((CHUNK_ELEMS,), jnp.float32),
            pltpu.VMEM((CHUNK_ELEMS,), jnp.float32),
            pltpu.VMEM((NUM_QUESTIONS,), jnp.int32),
            pltpu.VMEM((NUM_QUESTIONS,), jnp.int32),
            pltpu.VMEM((NUM_QUESTIONS,), jnp.float32),
            pltpu.VMEM((NUM_QUESTIONS * NUM_CONCEPTS,), jnp.float32),
            pltpu.SemaphoreType.DMA,
            pltpu.SemaphoreType.DMA,
            pltpu.SemaphoreType.DMA,
            pltpu.SemaphoreType.DMA,
        ],
    )(_fe_body)
    out_flat = run(A.reshape(-1), D.reshape(-1), concepts_row, concepts_col)
    return out_flat.reshape(NUM_STUDENTS, NUM_QUESTIONS)


TC_BLOCK = 1000


def _tc_body(col_ref, row_ref, D_ref, a_ref, out_ref):
    iota = lax.broadcasted_iota(jnp.int32, (NUM_CONCEPTS, NUM_QUESTIONS), 0)
    P = jnp.where(iota == col_ref[...], 1.0, 0.0).astype(jnp.float32)
    a = jnp.dot(a_ref[...], P, precision=lax.Precision.HIGHEST)
    DP = jnp.dot(D_ref[...], P, precision=lax.Precision.HIGHEST)
    M1 = jnp.where(iota == row_ref[...], 1.0, 0.0).astype(jnp.float32)
    d = jnp.sum(M1 * DP, axis=0, keepdims=True)
    out_ref[...] = jnp.maximum(1.0 - jnp.exp((d - a) * jnp.float32(L)),
                               jnp.float32(GUESS_PROB))


def _tc_forward(A, D, concepts_row, concepts_col, n_rows, row_offset=0):
    n_blocks = n_rows // TC_BLOCK
    blk0 = row_offset // TC_BLOCK
    col2 = concepts_col.reshape(1, NUM_QUESTIONS)
    row2 = concepts_row.reshape(1, NUM_QUESTIONS)
    return pl.pallas_call(
        _tc_body,
        grid=(n_blocks,),
        in_specs=[
            pl.BlockSpec((1, NUM_QUESTIONS), lambda i: (0, 0)),
            pl.BlockSpec((1, NUM_QUESTIONS), lambda i: (0, 0)),
            pl.BlockSpec((NUM_CONCEPTS, NUM_QUESTIONS), lambda i: (0, 0)),
            pl.BlockSpec((TC_BLOCK, NUM_QUESTIONS),
                         lambda i: (i + blk0, 0)),
        ],
        out_specs=pl.BlockSpec((TC_BLOCK, NUM_QUESTIONS), lambda i: (i, 0)),
        out_shape=jax.ShapeDtypeStruct((n_rows, NUM_QUESTIONS), jnp.float32),
    )(col2, row2, D, A)


@jax.jit
def _tc_model(A, D, concepts_row, concepts_col):
    return _tc_forward(A, D, concepts_row, concepts_col, NUM_STUDENTS)


def kernel(x, A, D, concepts_row, concepts_col):
    del x
    return _tc_model(A, D, concepts_row, concepts_col)


# speedup vs baseline: 1.4873x; 1.4873x over previous
import functools

import jax
import jax.numpy as jnp
from jax import lax
from jax.experimental import pallas as pl
from jax.experimental.pallas import tpu as pltpu
from jax.experimental.pallas import tpu_sc as plsc

NUM_STUDENTS = 100000
NUM_QUESTIONS = 128
NUM_CONCEPTS = 128
GUESS_PROB = 0.2
L = 10.0

LANES = 16
NUM_CORES = 2
NUM_SUBCORES = 16
NUM_WORKERS = NUM_CORES * NUM_SUBCORES
ROWS_PER_WORKER = NUM_STUDENTS // NUM_WORKERS
CHUNK = 125
NUM_CHUNKS = ROWS_PER_WORKER // CHUNK
NUM_PAIRS = NUM_CHUNKS // 2
GROUPS = NUM_QUESTIONS // LANES
CHUNK_ELEMS = CHUNK * NUM_QUESTIONS


def _fe_body(A_hbm, D_hbm, row_hbm, col_hbm, out_hbm,
             in0, in1, out0, out1, col_v, row_v, d_v, D_v,
             si0, si1, so0, so1):
    wid = lax.axis_index("s") * NUM_CORES + lax.axis_index("c")
    base = wid * ROWS_PER_WORKER

    def a_slice(ci):
        return A_hbm.at[pl.ds((base + ci * CHUNK) * NUM_QUESTIONS,
                              CHUNK_ELEMS)]

    def o_slice(ci):
        return out_hbm.at[pl.ds((base + ci * CHUNK) * NUM_QUESTIONS,
                                CHUNK_ELEMS)]

    def start_in(ci, buf, sem):
        pltpu.async_copy(a_slice(ci), buf, sem)

    def wait_in(ci, buf, sem):
        pltpu.make_async_copy(a_slice(ci), buf, sem).wait()

    def start_out(ci, buf, sem):
        pltpu.async_copy(buf, o_slice(ci), sem)

    def wait_out(ci, buf, sem):
        pltpu.make_async_copy(buf, o_slice(ci), sem).wait()

    start_in(0, in0, si0)

    pltpu.sync_copy(col_hbm, col_v)
    pltpu.sync_copy(row_hbm, row_v)
    pltpu.sync_copy(D_hbm, D_v)

    for g in range(GROUPS):
        sl = pl.ds(g * LANES, LANES)
        d_v[sl] = plsc.load_gather(
            D_v, [row_v[sl] * NUM_CONCEPTS + col_v[sl]])

    col_g = [col_v[pl.ds(g * LANES, LANES)] for g in range(GROUPS)]
    d10_g = [d_v[pl.ds(g * LANES, LANES)] * jnp.float32(L)
             for g in range(GROUPS)]

    def compute(in_buf, out_buf):
        @plsc.parallel_loop(0, CHUNK, unroll=4)
        def row_body(s):
            s_base = s * NUM_QUESTIONS
            for g in range(GROUPS):
                a = plsc.load_gather(in_buf, [col_g[g] + s_base])
                t = jnp.exp(a * jnp.float32(-L) + d10_g[g])
                out_buf[pl.ds(s_base + g * LANES, LANES)] = jnp.maximum(
                    1.0 - t, jnp.float32(GUESS_PROB))

    start_in(1, in1, si1)
    wait_in(0, in0, si0)
    compute(in0, out0)
    start_out(0, out0, so0)
    start_in(2, in0, si0)
    wait_in(1, in1, si1)
    compute(in1, out1)
    start_out(1, out1, so1)

    def pair_body(k, carry):
        c0 = 2 * k
        c1 = c0 + 1
        start_in(c1, in1, si1)
        wait_in(c0, in0, si0)
        wait_out(c0 - 2, out0, so0)
        compute(in0, out0)
        start_out(c0, out0, so0)
        start_in(c0 + 2, in0, si0)
        wait_in(c1, in1, si1)
        wait_out(c1 - 2, out1, so1)
        compute(in1, out1)
        start_out(c1, out1, so1)
        return carry

    lax.fori_loop(1, NUM_PAIRS, pair_body, 0)

    last = NUM_CHUNKS - 1
    wait_in(last, in0, si0)
    wait_out(last - 2, out0, so0)
    compute(in0, out0)
    start_out(last, out0, so0)
    wait_out(last - 1, out1, so1)
    wait_out(last, out0, so0)


@jax.jit
def _fe_model(A, D, concepts_row, concepts_col):
    mesh = plsc.VectorSubcoreMesh(core_axis_name="c", subcore_axis_name="s")
    run = functools.partial(
        pl.kernel,
        mesh=mesh,
        compiler_params=pltpu.CompilerParams(needs_layout_passes=False),
        out_type=jax.ShapeDtypeStruct((NUM_STUDENTS * NUM_QUESTIONS,),
                                      jnp.float32),
        scratch_types=[
            pltpu.VMEM((CHUNK_ELEMS,), jnp.float32),
            pltpu.VMEM((CHUNK_ELEMS,), jnp.float32),
            pltpu.VMEM((CHUNK_ELEMS,), jnp.float32),
            pltpu.VMEM((CHUNK_ELEMS,), jnp.float32),
            pltpu.VMEM((NUM_QUESTIONS,), jnp.int32),
            pltpu.VMEM((NUM_QUESTIONS,), jnp.int32),
            pltpu.VMEM((NUM_QUESTIONS,), jnp.float32),
            pltpu.VMEM((NUM_QUESTIONS * NUM_CONCEPTS,), jnp.float32),
            pltpu.SemaphoreType.DMA,
            pltpu.SemaphoreType.DMA,
            pltpu.SemaphoreType.DMA,
            pltpu.SemaphoreType.DMA,
        ],
    )(_fe_body)
    out_flat = run(A.reshape(-1), D.reshape(-1), concepts_row, concepts_col)
    return out_flat.reshape(NUM_STUDENTS, NUM_QUESTIONS)


TC_BLOCK = 1000


def _tc_body(col_ref, row_ref, D_ref, a_ref, out_ref):
    cb = jnp.broadcast_to(col_ref[...], (TC_BLOCK, NUM_QUESTIONS))
    a = jnp.take_along_axis(a_ref[...], cb, axis=1)
    Dg = jnp.take_along_axis(
        D_ref[...],
        jnp.broadcast_to(col_ref[...], (NUM_CONCEPTS, NUM_QUESTIONS)),
        axis=1)
    iota = lax.broadcasted_iota(jnp.int32, (NUM_CONCEPTS, NUM_QUESTIONS), 0)
    M1 = jnp.where(iota == row_ref[...], 1.0, 0.0).astype(jnp.float32)
    d = jnp.sum(M1 * Dg, axis=0, keepdims=True)
    out_ref[...] = jnp.maximum(1.0 - jnp.exp((d - a) * jnp.float32(L)),
                               jnp.float32(GUESS_PROB))


def _tc_forward(A, D, concepts_row, concepts_col, n_rows, row_offset=0):
    n_blocks = n_rows // TC_BLOCK
    blk0 = row_offset // TC_BLOCK
    col2 = concepts_col.reshape(1, NUM_QUESTIONS)
    row2 = concepts_row.reshape(1, NUM_QUESTIONS)
    return pl.pallas_call(
        _tc_body,
        grid=(n_blocks,),
        in_specs=[
            pl.BlockSpec((1, NUM_QUESTIONS), lambda i: (0, 0)),
            pl.BlockSpec((1, NUM_QUESTIONS), lambda i: (0, 0)),
            pl.BlockSpec((NUM_CONCEPTS, NUM_QUESTIONS), lambda i: (0, 0)),
            pl.BlockSpec((TC_BLOCK, NUM_QUESTIONS),
                         lambda i: (i + blk0, 0)),
        ],
        out_specs=pl.BlockSpec((TC_BLOCK, NUM_QUESTIONS), lambda i: (i, 0)),
        out_shape=jax.ShapeDtypeStruct((n_rows, NUM_QUESTIONS), jnp.float32),
    )(col2, row2, D, A)


@jax.jit
def _tc_model(A, D, concepts_row, concepts_col):
    return _tc_forward(A, D, concepts_row, concepts_col, NUM_STUDENTS)


def kernel(x, A, D, concepts_row, concepts_col):
    del x
    return _tc_model(A, D, concepts_row, concepts_col)


# speedup vs baseline: 2.6251x; 1.7650x over previous
import functools

import jax
import jax.numpy as jnp
from jax import lax
from jax.experimental import pallas as pl
from jax.experimental.pallas import tpu as pltpu
from jax.experimental.pallas import tpu_sc as plsc

NUM_STUDENTS = 100000
NUM_QUESTIONS = 128
NUM_CONCEPTS = 128
GUESS_PROB = 0.2
L = 10.0

LANES = 16
NUM_CORES = 2
NUM_SUBCORES = 16
NUM_WORKERS = NUM_CORES * NUM_SUBCORES
ROWS_PER_WORKER = NUM_STUDENTS // NUM_WORKERS
CHUNK = 125
NUM_CHUNKS = ROWS_PER_WORKER // CHUNK
NUM_PAIRS = NUM_CHUNKS // 2
GROUPS = NUM_QUESTIONS // LANES
CHUNK_ELEMS = CHUNK * NUM_QUESTIONS


def _fe_body(A_hbm, D_hbm, row_hbm, col_hbm, out_hbm,
             in0, in1, out0, out1, col_v, row_v, d_v, D_v,
             si0, si1, so0, so1):
    wid = lax.axis_index("s") * NUM_CORES + lax.axis_index("c")
    base = wid * ROWS_PER_WORKER

    def a_slice(ci):
        return A_hbm.at[pl.ds((base + ci * CHUNK) * NUM_QUESTIONS,
                              CHUNK_ELEMS)]

    def o_slice(ci):
        return out_hbm.at[pl.ds((base + ci * CHUNK) * NUM_QUESTIONS,
                                CHUNK_ELEMS)]

    def start_in(ci, buf, sem):
        pltpu.async_copy(a_slice(ci), buf, sem)

    def wait_in(ci, buf, sem):
        pltpu.make_async_copy(a_slice(ci), buf, sem).wait()

    def start_out(ci, buf, sem):
        pltpu.async_copy(buf, o_slice(ci), sem)

    def wait_out(ci, buf, sem):
        pltpu.make_async_copy(buf, o_slice(ci), sem).wait()

    start_in(0, in0, si0)

    pltpu.sync_copy(col_hbm, col_v)
    pltpu.sync_copy(row_hbm, row_v)
    pltpu.sync_copy(D_hbm, D_v)

    for g in range(GROUPS):
        sl = pl.ds(g * LANES, LANES)
        d_v[sl] = plsc.load_gather(
            D_v, [row_v[sl] * NUM_CONCEPTS + col_v[sl]])

    col_g = [col_v[pl.ds(g * LANES, LANES)] for g in range(GROUPS)]
    d10_g = [d_v[pl.ds(g * LANES, LANES)] * jnp.float32(L)
             for g in range(GROUPS)]

    def compute(in_buf, out_buf):
        @plsc.parallel_loop(0, CHUNK, unroll=4)
        def row_body(s):
            s_base = s * NUM_QUESTIONS
            for g in range(GROUPS):
                a = plsc.load_gather(in_buf, [col_g[g] + s_base])
                t = jnp.exp(a * jnp.float32(-L) + d10_g[g])
                out_buf[pl.ds(s_base + g * LANES, LANES)] = jnp.maximum(
                    1.0 - t, jnp.float32(GUESS_PROB))

    start_in(1, in1, si1)
    wait_in(0, in0, si0)
    compute(in0, out0)
    start_out(0, out0, so0)
    start_in(2, in0, si0)
    wait_in(1, in1, si1)
    compute(in1, out1)
    start_out(1, out1, so1)

    def pair_body(k, carry):
        c0 = 2 * k
        c1 = c0 + 1
        start_in(c1, in1, si1)
        wait_in(c0, in0, si0)
        wait_out(c0 - 2, out0, so0)
        compute(in0, out0)
        start_out(c0, out0, so0)
        start_in(c0 + 2, in0, si0)
        wait_in(c1, in1, si1)
        wait_out(c1 - 2, out1, so1)
        compute(in1, out1)
        start_out(c1, out1, so1)
        return carry

    lax.fori_loop(1, NUM_PAIRS, pair_body, 0)

    last = NUM_CHUNKS - 1
    wait_in(last, in0, si0)
    wait_out(last - 2, out0, so0)
    compute(in0, out0)
    start_out(last, out0, so0)
    wait_out(last - 1, out1, so1)
    wait_out(last, out0, so0)


@jax.jit
def _fe_model(A, D, concepts_row, concepts_col):
    mesh = plsc.VectorSubcoreMesh(core_axis_name="c", subcore_axis_name="s")
    run = functools.partial(
        pl.kernel,
        mesh=mesh,
        compiler_params=pltpu.CompilerParams(needs_layout_passes=False),
        out_type=jax.ShapeDtypeStruct((NUM_STUDENTS * NUM_QUESTIONS,),
                                      jnp.float32),
        scratch_types=[
            pltpu.VMEM((CHUNK_ELEMS,), jnp.float32),
            pltpu.VMEM((CHUNK_ELEMS,), jnp.float32),
            pltpu.VMEM((CHUNK_ELEMS,), jnp.float32),
            pltpu.VMEM((CHUNK_ELEMS,), jnp.float32),
            pltpu.VMEM((NUM_QUESTIONS,), jnp.int32),
            pltpu.VMEM((NUM_QUESTIONS,), jnp.int32),
            pltpu.VMEM((NUM_QUESTIONS,), jnp.float32),
            pltpu.VMEM((NUM_QUESTIONS * NUM_CONCEPTS,), jnp.float32),
            pltpu.SemaphoreType.DMA,
            pltpu.SemaphoreType.DMA,
            pltpu.SemaphoreType.DMA,
            pltpu.SemaphoreType.DMA,
        ],
    )(_fe_body)
    out_flat = run(A.reshape(-1), D.reshape(-1), concepts_row, concepts_col)
    return out_flat.reshape(NUM_STUDENTS, NUM_QUESTIONS)


TC_BLOCK = 4000


def _tc_body(col_ref, row_ref, D_ref, a_ref, out_ref):
    cb = jnp.broadcast_to(col_ref[...], (TC_BLOCK, NUM_QUESTIONS))
    a = jnp.take_along_axis(a_ref[...], cb, axis=1)
    Dg = jnp.take_along_axis(
        D_ref[...],
        jnp.broadcast_to(col_ref[...], (NUM_CONCEPTS, NUM_QUESTIONS)),
        axis=1)
    iota = lax.broadcasted_iota(jnp.int32, (NUM_CONCEPTS, NUM_QUESTIONS), 0)
    M1 = jnp.where(iota == row_ref[...], 1.0, 0.0).astype(jnp.float32)
    d = jnp.sum(M1 * Dg, axis=0, keepdims=True)
    out_ref[...] = jnp.maximum(1.0 - jnp.exp((d - a) * jnp.float32(L)),
                               jnp.float32(GUESS_PROB))


def _tc_forward(A, D, concepts_row, concepts_col, n_rows, row_offset=0):
    n_blocks = n_rows // TC_BLOCK
    blk0 = row_offset // TC_BLOCK
    col2 = concepts_col.reshape(1, NUM_QUESTIONS)
    row2 = concepts_row.reshape(1, NUM_QUESTIONS)
    return pl.pallas_call(
        _tc_body,
        grid=(n_blocks,),
        in_specs=[
            pl.BlockSpec((1, NUM_QUESTIONS), lambda i: (0, 0)),
            pl.BlockSpec((1, NUM_QUESTIONS), lambda i: (0, 0)),
            pl.BlockSpec((NUM_CONCEPTS, NUM_QUESTIONS), lambda i: (0, 0)),
            pl.BlockSpec((TC_BLOCK, NUM_QUESTIONS),
                         lambda i: (i + blk0, 0)),
        ],
        out_specs=pl.BlockSpec((TC_BLOCK, NUM_QUESTIONS), lambda i: (i, 0)),
        out_shape=jax.ShapeDtypeStruct((n_rows, NUM_QUESTIONS), jnp.float32),
    )(col2, row2, D, A)


@jax.jit
def _tc_model(A, D, concepts_row, concepts_col):
    return _tc_forward(A, D, concepts_row, concepts_col, NUM_STUDENTS)


def kernel(x, A, D, concepts_row, concepts_col):
    del x
    return _tc_model(A, D, concepts_row, concepts_col)
